# baseline (speedup 1.0000x reference)
import functools

import jax
import jax.numpy as jnp
from jax import lax
from jax.experimental import pallas as pl
from jax.experimental.pallas import tpu as pltpu, tpu_sc as plsc

EPS = 1e-12
N, D, K = 16384, 256, 1024
DH = D // 2
NC, NS = 2, 16
NW = NC * NS
ROWS_PER_W = N // NW
CHUNK = 128
NCHUNK = ROWS_PER_W // CHUNK
BIG = 1e30


def _sc_mesh():
    return plsc.VectorSubcoreMesh(core_axis_name="c", subcore_axis_name="s",
                                  num_cores=NC, num_subcores=NS)


def _norm_body(z_ref, o_ref):
    zb = z_ref[...]
    ss = jnp.sum(zb * zb, axis=1, keepdims=True)
    o_ref[...] = zb / jnp.sqrt(jnp.maximum(ss, 1e-24))


def _normalize(z):
    blk = 4096
    return pl.pallas_call(
        _norm_body,
        grid=(N // blk,),
        in_specs=[pl.BlockSpec((blk, D), lambda i: (i, 0))],
        out_specs=pl.BlockSpec((blk, D), lambda i: (i, 0)),
        out_shape=jax.ShapeDtypeStruct((N, D), jnp.float32),
    )(z)


def _sc_scatter(z_hat, y):
    @functools.partial(
        pl.kernel,
        out_type=(
            jax.ShapeDtypeStruct((NC, 2, K, DH), jnp.float32),
            jax.ShapeDtypeStruct((NC, K, DH), jnp.float32),
        ),
        mesh=_sc_mesh(),
        scratch_types=[
            pltpu.VMEM((CHUNK, DH), jnp.float32),
            pltpu.VMEM((CHUNK, DH), jnp.float32),
            pltpu.VMEM((CHUNK, DH), jnp.float32),
            pltpu.VMEM((CHUNK, DH), jnp.float32),
            pltpu.VMEM((CHUNK, DH), jnp.float32),
            pltpu.VMEM((CHUNK,), jnp.int32),
            pltpu.VMEM((CHUNK,), jnp.int32),
            pltpu.VMEM((CHUNK,), jnp.int32),
            pltpu.VMEM((CHUNK,), jnp.int32),
            pltpu.VMEM((K // NS, DH), jnp.float32),
            pltpu.VMEM_SHARED((K, DH), jnp.float32),
            pltpu.VMEM_SHARED((K, DH), jnp.float32),
            pltpu.VMEM_SHARED((K, DH), jnp.float32),
        ] + [pltpu.SemaphoreType.DMA] * 11,
    )
    def k(z_hbm, y_hbm, s_out, c_out,
          ra0, ra1, rb0, rb1, ones_v, ix0, ix1, ix2, ix3, zbuf, sa, sb, sc,
          sia, sib, sic, sid_, siza0, siza1, sizb0, sizb1, ssa, ssb, sso):
        cid = lax.axis_index("c")
        sid = lax.axis_index("s")
        stripe = K // NS
        ras, rbs = [ra0, ra1], [rb0, rb1]
        ixs = [ix0, ix1, ix2, ix3]
        isem = [sia, sib, sic, sid_]
        zsem_a, zsem_b = [siza0, siza1], [sizb0, sizb1]
        base = (sid * NC + cid) * ROWS_PER_W

        def start_idx(kk):
            b = base + kk * CHUNK
            return pltpu.async_copy(y_hbm.at[pl.ds(b, CHUNK)],
                                    ixs[kk], isem[kk])

        def start_z(kk):
            s = kk % 2
            b = base + kk * CHUNK
            da = pltpu.async_copy(z_hbm.at[pl.ds(b, CHUNK), pl.ds(0, DH)],
                                  ras[s], zsem_a[s])
            db = pltpu.async_copy(z_hbm.at[pl.ds(b, CHUNK), pl.ds(DH, DH)],
                                  rbs[s], zsem_b[s])
            return da, db

        idx_d = [start_idx(kk) for kk in range(NCHUNK)]
        z_d = [None] * NCHUNK
        z_d[0] = start_z(0)

        def zb_body(i, c):
            for j in range(DH // 16):
                zbuf[i, pl.ds(j * 16, 16)] = jnp.zeros((16,), jnp.float32)
            return c
        lax.fori_loop(0, stripe, zb_body, 0)

        def on_body(i, c):
            for j in range(DH // 16):
                ones_v[i, pl.ds(j * 16, 16)] = jnp.ones((16,), jnp.float32)
            return c
        lax.fori_loop(0, CHUNK, on_body, 0)

        pltpu.sync_copy(zbuf, sa.at[pl.ds(sid * stripe, stripe)])
        pltpu.sync_copy(zbuf, sb.at[pl.ds(sid * stripe, stripe)])
        pltpu.sync_copy(zbuf, sc.at[pl.ds(sid * stripe, stripe)])
        plsc.subcore_barrier()

        scat_d = [None] * NCHUNK
        for kk in range(NCHUNK):
            s = kk % 2
            z_d[kk][0].wait()
            z_d[kk][1].wait()
            if kk >= 1:
                for dd in scat_d[kk - 1]:
                    dd.wait()
            if kk + 1 < NCHUNK:
                z_d[kk + 1] = start_z(kk + 1)
            idx_d[kk].wait()
            scat_d[kk] = (
                pltpu.async_copy(ras[s], sa.at[ixs[kk]], ssa, add=True),
                pltpu.async_copy(rbs[s], sb.at[ixs[kk]], ssb, add=True),
                pltpu.async_copy(ones_v, sc.at[ixs[kk]], sso, add=True),
            )
        for dd in scat_d[NCHUNK - 1]:
            dd.wait()
        plsc.subcore_barrier()
        pltpu.sync_copy(sa.at[pl.ds(sid * stripe, stripe)],
                        s_out.at[cid, 0, pl.ds(sid * stripe, stripe)])
        pltpu.sync_copy(sb.at[pl.ds(sid * stripe, stripe)],
                        s_out.at[cid, 1, pl.ds(sid * stripe, stripe)])
        pltpu.sync_copy(sc.at[pl.ds(sid * stripe, stripe)],
                        c_out.at[cid, pl.ds(sid * stripe, stripe)])

    return k(z_hat, y)


def _stats_body(sp_ref, cp_ref, a_ref, g_ref):
    s_lo = sp_ref[0, 0] + sp_ref[1, 0]
    s_hi = sp_ref[0, 1] + sp_ref[1, 1]
    S = jnp.concatenate([s_lo, s_hi], axis=1)
    counts = (cp_ref[0] + cp_ref[1])[:, 0:1]
    present = counts > 0.5
    mu = S / jnp.maximum(counts, 1.0)
    q = jnp.sum(mu * mu, axis=1, keepdims=True)
    ones = jnp.ones_like(q)
    absent = jnp.where(present, 0.0, BIG)
    ml = jnp.concatenate([mu, q, ones, absent, ones], axis=1)
    mr = jnp.concatenate([-2.0 * mu, ones, q, ones, absent], axis=1)
    d2 = lax.dot_general(ml, mr, (((1,), (1,)), ((), ())),
                         preferred_element_type=jnp.float32)
    ri = lax.broadcasted_iota(jnp.int32, (K, K), 0)
    ci = lax.broadcasted_iota(jnp.int32, (K, K), 1)
    d2 = jnp.where(ri == ci, jnp.inf, d2)
    m2 = jnp.min(d2, axis=1, keepdims=True)
    m = jnp.sqrt(jnp.maximum(m2, 0.0))
    r = jnp.sqrt(jnp.where(present, jnp.maximum(1.0 - q, 0.0), 0.0) + EPS)
    gamma = jnp.maximum(m / (2.0 * r + EPS), 1.0)
    npresent = jnp.sum(jnp.where(present, 1.0, 0.0))
    gamma = jnp.where((npresent >= 2.0) & present, gamma, 1.0)
    a_ref[...] = (1.0 - gamma) * mu
    ci_p = lax.broadcasted_iota(jnp.int32, (K // 8, K), 1)
    ri_p = lax.broadcasted_iota(jnp.int32, (K // 8, K), 0)
    p_sel = jnp.where((ci_p >> 3) == ri_p, 1.0, 0.0)
    rc_w = lax.broadcasted_iota(jnp.int32, (K, 128), 0)
    cc_w = lax.broadcasted_iota(jnp.int32, (K, 128), 1)
    w_sel = jnp.where((cc_w >> 4) == (rc_w & 7), gamma, 0.0)
    g_ref[...] = lax.dot_general(p_sel, w_sel, (((1,), (0,)), ((), ())),
                                 preferred_element_type=jnp.float32)


def _stats(s_parts, c_parts):
    return pl.pallas_call(
        _stats_body,
        out_shape=(
            jax.ShapeDtypeStruct((K, D), jnp.float32),
            jax.ShapeDtypeStruct((K // 8, 128), jnp.float32),
        ),
    )(s_parts, c_parts)


CHUNK4 = 64
NCHUNK4 = ROWS_PER_W // CHUNK4


def _sc_combine(z_hat, y, a_tab, g_tab):
    @functools.partial(
        pl.kernel,
        out_type=jax.ShapeDtypeStruct((N, D), jnp.float32),
        mesh=_sc_mesh(),
        scratch_types=[
            pltpu.VMEM((CHUNK4, D), jnp.float32),
            pltpu.VMEM((CHUNK4, D), jnp.float32),
            pltpu.VMEM((CHUNK4, D), jnp.float32),
            pltpu.VMEM((CHUNK4, D), jnp.float32),
            pltpu.VMEM((CHUNK4, D), jnp.float32),
            pltpu.VMEM((CHUNK4, D), jnp.float32),
            pltpu.VMEM((CHUNK4 + 16,), jnp.int32),
            pltpu.VMEM((CHUNK4 + 16,), jnp.int32),
            pltpu.VMEM((CHUNK4 + 16,), jnp.int32),
            pltpu.VMEM((CHUNK4 + 16,), jnp.int32),
            pltpu.VMEM((K // 8, 128), jnp.float32),
        ] + [pltpu.SemaphoreType.DMA] * 14,
    )
    def k(z_hbm, y_hbm, a_hbm, g_hbm, out_hbm,
          zr0, zr1, zr2, ar0, ar1, ar2, ix0, ix1, ix2, ix3, gtab,
          sz0, sz1, sz2, st0, st1, st2, si0, si1, si2, si3,
          so0, so1, so2, sg):
        cid = lax.axis_index("c")
        sid = lax.axis_index("s")
        base = (sid * NC + cid) * ROWS_PER_W
        zrs, ars = [zr0, zr1, zr2], [ar0, ar1, ar2]
        ixs, isem = [ix0, ix1, ix2, ix3], [si0, si1, si2, si3]
        zsem, tsem = [sz0, sz1, sz2], [st0, st1, st2]
        osem = [so0, so1, so2]

        def start_idx(kk):
            b = base + kk * CHUNK4
            return pltpu.async_copy(y_hbm.at[pl.ds(b, CHUNK4)],
                                    ixs[kk % 4].at[pl.ds(0, CHUNK4)],
                                    isem[kk % 4])

        def start_zt(kk):
            s = kk % 3
            b = base + kk * CHUNK4
            dz = pltpu.async_copy(z_hbm.at[pl.ds(b, CHUNK4)], zrs[s], zsem[s])
            dt = pltpu.async_copy(a_hbm.at[ixs[kk % 4].at[pl.ds(0, CHUNK4)]],
                                  ars[s], tsem[s])
            return dz, dt

        g_d = pltpu.async_copy(g_hbm, gtab, sg)
        idx_d = [None] * NCHUNK4
        zt_d = [None] * NCHUNK4
        out_d = [None] * NCHUNK4
        idx_d[0] = start_idx(0)
        idx_d[0].wait()
        zt_d[0] = start_zt(0)
        if NCHUNK4 > 1:
            idx_d[1] = start_idx(1)
            idx_d[1].wait()
            zt_d[1] = start_zt(1)
        if NCHUNK4 > 2:
            idx_d[2] = start_idx(2)
        g_d.wait()
        for kk in range(NCHUNK4):
            s = kk % 3
            zt_d[kk][0].wait()
            zt_d[kk][1].wait()
            if kk + 2 < NCHUNK4:
                idx_d[kk + 2].wait()
                if kk >= 1:
                    out_d[kk - 1].wait()
                zt_d[kk + 2] = start_zt(kk + 2)
            if kk + 3 < NCHUNK4:
                idx_d[kk + 3] = start_idx(kk + 3)
            zr_s, ar_s, ix = zrs[s], ars[s], ixs[kk % 4]

            @plsc.parallel_loop(0, CHUNK4, step=1, unroll=4)
            def row_body(r):
                yv = ix[pl.ds(r, 16)][0]
                g = gtab[yv >> 3, pl.ds((yv & 7) * 16, 16)]
                for j in range(D // 16):
                    sl = pl.ds(j * 16, 16)
                    zr_s[r, sl] = ar_s[r, sl] + g * zr_s[r, sl]

            b = base + kk * CHUNK4
            out_d[kk] = pltpu.async_copy(zr_s, out_hbm.at[pl.ds(b, CHUNK4)],
                                         osem[s])
        out_d[NCHUNK4 - 3].wait()
        out_d[NCHUNK4 - 2].wait()
        out_d[NCHUNK4 - 1].wait()

    return k(z_hat, y, a_tab, g_tab)


def kernel(z, y):
    yi = y.astype(jnp.int32)
    z_hat = _normalize(z)
    s_parts, c_parts = _sc_scatter(z_hat, yi)
    a_tab, g_tab = _stats(s_parts, c_parts)
    return _sc_combine(z_hat, yi, a_tab, g_tab)


# speedup vs baseline: 1.0331x; 1.0331x over previous
import functools

import jax
import jax.numpy as jnp
from jax import lax
from jax.experimental import pallas as pl
from jax.experimental.pallas import tpu as pltpu, tpu_sc as plsc

EPS = 1e-12
N, D, K = 16384, 256, 1024
DH = D // 2
NC, NS = 2, 16
NW = NC * NS
ROWS_PER_W = N // NW
CHUNK = 128
NCHUNK = ROWS_PER_W // CHUNK
BIG = 1e30


def _sc_mesh():
    return plsc.VectorSubcoreMesh(core_axis_name="c", subcore_axis_name="s",
                                  num_cores=NC, num_subcores=NS)


def _norm_body(z_ref, o_ref):
    zb = z_ref[...]
    ss = jnp.sum(zb * zb, axis=1, keepdims=True)
    o_ref[...] = zb / jnp.sqrt(jnp.maximum(ss, 1e-24))


def _normalize(z):
    blk = 4096
    return pl.pallas_call(
        _norm_body,
        grid=(N // blk,),
        in_specs=[pl.BlockSpec((blk, D), lambda i: (i, 0))],
        out_specs=pl.BlockSpec((blk, D), lambda i: (i, 0)),
        out_shape=jax.ShapeDtypeStruct((N, D), jnp.float32),
    )(z)


def _sc_scatter(z_hat, y):
    @functools.partial(
        pl.kernel,
        out_type=(
            jax.ShapeDtypeStruct((NC, 2, K, DH), jnp.float32),
            jax.ShapeDtypeStruct((NC, K, DH), jnp.float32),
        ),
        mesh=_sc_mesh(),
        scratch_types=[
            pltpu.VMEM((CHUNK, DH), jnp.float32),
            pltpu.VMEM((CHUNK, DH), jnp.float32),
            pltpu.VMEM((CHUNK, DH), jnp.float32),
            pltpu.VMEM((CHUNK, DH), jnp.float32),
            pltpu.VMEM((CHUNK, DH), jnp.float32),
            pltpu.VMEM((CHUNK,), jnp.int32),
            pltpu.VMEM((CHUNK,), jnp.int32),
            pltpu.VMEM((CHUNK,), jnp.int32),
            pltpu.VMEM((CHUNK,), jnp.int32),
            pltpu.VMEM((K // NS, DH), jnp.float32),
            pltpu.VMEM_SHARED((K, DH), jnp.float32),
            pltpu.VMEM_SHARED((K, DH), jnp.float32),
            pltpu.VMEM_SHARED((K, DH), jnp.float32),
        ] + [pltpu.SemaphoreType.DMA] * 11,
    )
    def k(z_hbm, y_hbm, s_out, c_out,
          ra0, ra1, rb0, rb1, ones_v, ix0, ix1, ix2, ix3, zbuf, sa, sb, sc,
          sia, sib, sic, sid_, siza0, siza1, sizb0, sizb1, ssa, ssb, sso):
        cid = lax.axis_index("c")
        sid = lax.axis_index("s")
        stripe = K // NS
        ras, rbs = [ra0, ra1], [rb0, rb1]
        ixs = [ix0, ix1, ix2, ix3]
        isem = [sia, sib, sic, sid_]
        zsem_a, zsem_b = [siza0, siza1], [sizb0, sizb1]
        base = (sid * NC + cid) * ROWS_PER_W

        def start_idx(kk):
            b = base + kk * CHUNK
            return pltpu.async_copy(y_hbm.at[pl.ds(b, CHUNK)],
                                    ixs[kk], isem[kk])

        def start_z(kk):
            s = kk % 2
            b = base + kk * CHUNK
            da = pltpu.async_copy(z_hbm.at[pl.ds(b, CHUNK), pl.ds(0, DH)],
                                  ras[s], zsem_a[s])
            db = pltpu.async_copy(z_hbm.at[pl.ds(b, CHUNK), pl.ds(DH, DH)],
                                  rbs[s], zsem_b[s])
            return da, db

        idx_d = [start_idx(kk) for kk in range(NCHUNK)]
        z_d = [None] * NCHUNK
        z_d[0] = start_z(0)

        def zb_body(i, c):
            for j in range(DH // 16):
                zbuf[i, pl.ds(j * 16, 16)] = jnp.zeros((16,), jnp.float32)
            return c
        lax.fori_loop(0, stripe, zb_body, 0)

        def on_body(i, c):
            for j in range(DH // 16):
                ones_v[i, pl.ds(j * 16, 16)] = jnp.ones((16,), jnp.float32)
            return c
        lax.fori_loop(0, CHUNK, on_body, 0)

        pltpu.sync_copy(zbuf, sa.at[pl.ds(sid * stripe, stripe)])
        pltpu.sync_copy(zbuf, sb.at[pl.ds(sid * stripe, stripe)])
        pltpu.sync_copy(zbuf, sc.at[pl.ds(sid * stripe, stripe)])
        plsc.subcore_barrier()

        scat_d = [None] * NCHUNK
        for kk in range(NCHUNK):
            s = kk % 2
            z_d[kk][0].wait()
            z_d[kk][1].wait()
            if kk >= 1:
                for dd in scat_d[kk - 1]:
                    dd.wait()
            if kk + 1 < NCHUNK:
                z_d[kk + 1] = start_z(kk + 1)
            idx_d[kk].wait()
            scat_d[kk] = (
                pltpu.async_copy(ras[s], sa.at[ixs[kk]], ssa, add=True),
                pltpu.async_copy(rbs[s], sb.at[ixs[kk]], ssb, add=True),
                pltpu.async_copy(ones_v, sc.at[ixs[kk]], sso, add=True),
            )
        for dd in scat_d[NCHUNK - 1]:
            dd.wait()
        plsc.subcore_barrier()
        pltpu.sync_copy(sa.at[pl.ds(sid * stripe, stripe)],
                        s_out.at[cid, 0, pl.ds(sid * stripe, stripe)])
        pltpu.sync_copy(sb.at[pl.ds(sid * stripe, stripe)],
                        s_out.at[cid, 1, pl.ds(sid * stripe, stripe)])
        pltpu.sync_copy(sc.at[pl.ds(sid * stripe, stripe)],
                        c_out.at[cid, pl.ds(sid * stripe, stripe)])

    return k(z_hat, y)


def _stats_body(sp_ref, cp_ref, a_ref, g_ref):
    s_lo = sp_ref[0, 0] + sp_ref[1, 0]
    s_hi = sp_ref[0, 1] + sp_ref[1, 1]
    S = jnp.concatenate([s_lo, s_hi], axis=1)
    counts = (cp_ref[0] + cp_ref[1])[:, 0:1]
    present = counts > 0.5
    mu = S / jnp.maximum(counts, 1.0)
    q = jnp.sum(mu * mu, axis=1, keepdims=True)
    ones = jnp.ones_like(q)
    absent = jnp.where(present, 0.0, BIG)
    ml = jnp.concatenate([mu, q, ones, absent, ones], axis=1)
    mr = jnp.concatenate([-2.0 * mu, ones, q, ones, absent], axis=1)
    d2 = lax.dot_general(ml, mr, (((1,), (1,)), ((), ())),
                         preferred_element_type=jnp.float32)
    ri = lax.broadcasted_iota(jnp.int32, (K, K), 0)
    ci = lax.broadcasted_iota(jnp.int32, (K, K), 1)
    d2 = jnp.where(ri == ci, jnp.inf, d2)
    m2 = jnp.min(d2, axis=1, keepdims=True)
    m = jnp.sqrt(jnp.maximum(m2, 0.0))
    r = jnp.sqrt(jnp.where(present, jnp.maximum(1.0 - q, 0.0), 0.0) + EPS)
    gamma = jnp.maximum(m / (2.0 * r + EPS), 1.0)
    npresent = jnp.sum(jnp.where(present, 1.0, 0.0))
    gamma = jnp.where((npresent >= 2.0) & present, gamma, 1.0)
    a_ref[...] = (1.0 - gamma) * mu
    ci_p = lax.broadcasted_iota(jnp.int32, (K // 8, K), 1)
    ri_p = lax.broadcasted_iota(jnp.int32, (K // 8, K), 0)
    p_sel = jnp.where((ci_p >> 3) == ri_p, 1.0, 0.0)
    rc_w = lax.broadcasted_iota(jnp.int32, (K, 128), 0)
    cc_w = lax.broadcasted_iota(jnp.int32, (K, 128), 1)
    w_sel = jnp.where((cc_w >> 4) == (rc_w & 7), gamma, 0.0)
    g_ref[...] = lax.dot_general(p_sel, w_sel, (((1,), (0,)), ((), ())),
                                 preferred_element_type=jnp.float32)


def _stats(s_parts, c_parts):
    return pl.pallas_call(
        _stats_body,
        out_shape=(
            jax.ShapeDtypeStruct((K, D), jnp.float32),
            jax.ShapeDtypeStruct((K // 8, 128), jnp.float32),
        ),
    )(s_parts, c_parts)


CHUNK4 = 64
NCHUNK4 = ROWS_PER_W // CHUNK4


def _sc_combine(z_hat, y, a_tab, g_tab):
    @functools.partial(
        pl.kernel,
        out_type=jax.ShapeDtypeStruct((N, D), jnp.float32),
        mesh=_sc_mesh(),
        scratch_types=[
            pltpu.VMEM((CHUNK4, D), jnp.float32),
            pltpu.VMEM((CHUNK4, D), jnp.float32),
            pltpu.VMEM((CHUNK4, D), jnp.float32),
            pltpu.VMEM((CHUNK4, D), jnp.float32),
            pltpu.VMEM((CHUNK4, D), jnp.float32),
            pltpu.VMEM((CHUNK4, D), jnp.float32),
            pltpu.VMEM((CHUNK4 + 16,), jnp.int32),
            pltpu.VMEM((CHUNK4 + 16,), jnp.int32),
            pltpu.VMEM((CHUNK4 + 16,), jnp.int32),
            pltpu.VMEM((CHUNK4 + 16,), jnp.int32),
            pltpu.VMEM((K // 8, 128), jnp.float32),
        ] + [pltpu.SemaphoreType.DMA] * 14,
    )
    def k(z_hbm, y_hbm, a_hbm, g_hbm, out_hbm,
          zr0, zr1, zr2, ar0, ar1, ar2, ix0, ix1, ix2, ix3, gtab,
          sz0, sz1, sz2, st0, st1, st2, si0, si1, si2, si3,
          so0, so1, so2, sg):
        cid = lax.axis_index("c")
        sid = lax.axis_index("s")
        base = (sid * NC + cid) * ROWS_PER_W
        zrs, ars = [zr0, zr1, zr2], [ar0, ar1, ar2]
        ixs, isem = [ix0, ix1, ix2, ix3], [si0, si1, si2, si3]
        zsem, tsem = [sz0, sz1, sz2], [st0, st1, st2]
        osem = [so0, so1, so2]

        def start_idx(kk):
            b = base + kk * CHUNK4
            return pltpu.async_copy(y_hbm.at[pl.ds(b, CHUNK4)],
                                    ixs[kk % 4].at[pl.ds(0, CHUNK4)],
                                    isem[kk % 4])

        def start_zt(kk):
            s = kk % 3
            b = base + kk * CHUNK4
            dz = pltpu.async_copy(z_hbm.at[pl.ds(b, CHUNK4)], zrs[s], zsem[s])
            dt = pltpu.async_copy(a_hbm.at[ixs[kk % 4].at[pl.ds(0, CHUNK4)]],
                                  ars[s], tsem[s])
            return dz, dt

        g_d = pltpu.async_copy(g_hbm, gtab, sg)
        idx_d = [None] * NCHUNK4
        zt_d = [None] * NCHUNK4
        out_d = [None] * NCHUNK4
        idx_d[0] = start_idx(0)
        idx_d[0].wait()
        zt_d[0] = start_zt(0)
        if NCHUNK4 > 1:
            idx_d[1] = start_idx(1)
            idx_d[1].wait()
            zt_d[1] = start_zt(1)
        if NCHUNK4 > 2:
            idx_d[2] = start_idx(2)
        g_d.wait()
        for kk in range(NCHUNK4):
            s = kk % 3
            zt_d[kk][0].wait()
            zt_d[kk][1].wait()
            if kk + 2 < NCHUNK4:
                idx_d[kk + 2].wait()
                if kk >= 1:
                    out_d[kk - 1].wait()
                zt_d[kk + 2] = start_zt(kk + 2)
            if kk + 3 < NCHUNK4:
                idx_d[kk + 3] = start_idx(kk + 3)
            zr_s, ar_s, ix = zrs[s], ars[s], ixs[kk % 4]

            @plsc.parallel_loop(0, CHUNK4, step=1, unroll=2)
            def row_body(r):
                yv = ix[pl.ds(r, 16)][0]
                g = gtab[yv >> 3, pl.ds((yv & 7) * 16, 16)]
                for j in range(D // 16):
                    sl = pl.ds(j * 16, 16)
                    zr_s[r, sl] = ar_s[r, sl] + g * zr_s[r, sl]

            b = base + kk * CHUNK4
            out_d[kk] = pltpu.async_copy(zr_s, out_hbm.at[pl.ds(b, CHUNK4)],
                                         osem[s])
        out_d[NCHUNK4 - 3].wait()
        out_d[NCHUNK4 - 2].wait()
        out_d[NCHUNK4 - 1].wait()

    return k(z_hat, y, a_tab, g_tab)


def kernel(z, y):
    yi = y.astype(jnp.int32)
    z_hat = _normalize(z)
    s_parts, c_parts = _sc_scatter(z_hat, yi)
    a_tab, g_tab = _stats(s_parts, c_parts)
    return _sc_combine(z_hat, yi, a_tab, g_tab)
